# Initial kernel scaffold; baseline (speedup 1.0000x reference)
import jax
import jax.numpy as jnp
from jax.experimental import pallas as pl


def kernel(x, edge_index, att_w, att_b):
    raise NotImplementedError("write your pallas kernel here")


# speedup vs baseline: 8.5472x; 8.5472x over previous
import functools

import jax
import jax.numpy as jnp
from jax import lax
from jax.experimental import pallas as pl
from jax.experimental.pallas import tpu as pltpu
from jax.experimental.pallas import tpu_sc as plsc

N_NODES = 10000
D_FEAT = 128
N_EDGES = 320000

NUM_CORES = 2
NUM_SUBCORES = 16
EDGES_PER_CORE = N_EDGES // NUM_CORES
EDGES_PER_TILE = EDGES_PER_CORE // NUM_SUBCORES
BLOCK_E = 80
NUM_BLOCKS = EDGES_PER_TILE // BLOCK_E
N_PAD = 10240
ROWS_PER_TILE = N_PAD // NUM_SUBCORES
ZROWS = 32


def _make_sc_kernel():
    mesh = plsc.VectorSubcoreMesh(core_axis_name="c", subcore_axis_name="s")

    @functools.partial(
        pl.kernel,
        mesh=mesh,
        out_type=jax.ShapeDtypeStruct((NUM_CORES * N_PAD, D_FEAT), jnp.float32),
        scratch_types=[
            pltpu.VMEM((BLOCK_E,), jnp.int32),
            pltpu.VMEM((BLOCK_E,), jnp.int32),
            pltpu.VMEM((BLOCK_E, D_FEAT), jnp.float32),
            pltpu.VMEM((ZROWS, D_FEAT), jnp.float32),
            pltpu.VMEM_SHARED((N_PAD, D_FEAT), jnp.float32),
        ],
    )
    def sc_segment_sum(src_hbm, dst_hbm, x_hbm, out_hbm,
                       src_v, dst_v, rows_v, zbuf, acc):
        cid = lax.axis_index("c")
        sid = lax.axis_index("s")
        row0 = sid * ROWS_PER_TILE

        zvec = jnp.zeros((16,), jnp.float32)

        def zfill(i, carry):
            r = i // (D_FEAT // 16)
            col = (i % (D_FEAT // 16)) * 16
            zbuf[r, pl.ds(col, 16)] = zvec
            return carry

        lax.fori_loop(0, ZROWS * (D_FEAT // 16), zfill, 0)

        def zcopy(j, carry):
            pltpu.sync_copy(zbuf, acc.at[pl.ds(row0 + j * ZROWS, ZROWS)])
            return carry

        lax.fori_loop(0, ROWS_PER_TILE // ZROWS, zcopy, 0)

        plsc.subcore_barrier()

        ebase = cid * EDGES_PER_CORE + sid * EDGES_PER_TILE

        def eblock(b, carry):
            off = ebase + b * BLOCK_E
            pltpu.sync_copy(src_hbm.at[pl.ds(off, BLOCK_E)], src_v)
            pltpu.sync_copy(dst_hbm.at[pl.ds(off, BLOCK_E)], dst_v)
            pltpu.sync_copy(x_hbm.at[src_v], rows_v)
            pltpu.sync_copy(rows_v, acc.at[dst_v], add=True)
            return carry

        lax.fori_loop(0, NUM_BLOCKS, eblock, 0)

        plsc.subcore_barrier()

        pltpu.sync_copy(
            acc.at[pl.ds(row0, ROWS_PER_TILE)],
            out_hbm.at[pl.ds(cid * N_PAD + row0, ROWS_PER_TILE)],
        )

    return sc_segment_sum


_SC_KERNEL = _make_sc_kernel()


def _add_body(a_ref, b_ref, o_ref):
    o_ref[...] = a_ref[...] + b_ref[...]


_ADD_BLOCK = 640
_NB_ADD = N_PAD // _ADD_BLOCK


def _combine(partials):
    summed = pl.pallas_call(
        _add_body,
        out_shape=jax.ShapeDtypeStruct((N_PAD, D_FEAT), jnp.float32),
        grid=(_NB_ADD,),
        in_specs=[
            pl.BlockSpec((_ADD_BLOCK, D_FEAT), lambda i: (i, 0)),
            pl.BlockSpec((_ADD_BLOCK, D_FEAT), lambda i: (i + _NB_ADD, 0)),
        ],
        out_specs=pl.BlockSpec((_ADD_BLOCK, D_FEAT), lambda i: (i, 0)),
    )(partials, partials)
    return summed[:N_NODES]


@jax.jit
def kernel(x, edge_index, att_w, att_b):
    del att_w, att_b
    src = edge_index[0].astype(jnp.int32)
    dst = edge_index[1].astype(jnp.int32)
    partials = _SC_KERNEL(src, dst, x.astype(jnp.float32))
    return _combine(partials)


# speedup vs baseline: 21.6467x; 2.5326x over previous
import functools

import jax
import jax.numpy as jnp
from jax import lax
from jax.experimental import pallas as pl
from jax.experimental.pallas import tpu as pltpu
from jax.experimental.pallas import tpu_sc as plsc

N_NODES = 10000
D_FEAT = 128
N_EDGES = 320000

NUM_CORES = 2
NUM_SUBCORES = 16
NUM_TILES = NUM_CORES * NUM_SUBCORES
EDGES_PER_TILE = N_EDGES // NUM_TILES
BLOCK_E = 80
NUM_BLOCKS = EDGES_PER_TILE // BLOCK_E
NBUF = 3
N_PAD = 10240
ROWS_PER_TILE = N_PAD // NUM_SUBCORES
ZROWS = 16


def _make_sc_kernel():
    mesh = plsc.VectorSubcoreMesh(core_axis_name="c", subcore_axis_name="s")

    @functools.partial(
        pl.kernel,
        mesh=mesh,
        out_type=jax.ShapeDtypeStruct((NUM_CORES * N_PAD, D_FEAT), jnp.float32),
        scratch_types=[
            pltpu.VMEM((EDGES_PER_TILE,), jnp.int32),
            [pltpu.VMEM((BLOCK_E, D_FEAT), jnp.float32) for _ in range(NBUF)],
            [pltpu.VMEM((BLOCK_E,), jnp.int32) for _ in range(NBUF)],
            pltpu.VMEM((ZROWS, D_FEAT), jnp.float32),
            pltpu.VMEM_SHARED((N_PAD, D_FEAT), jnp.float32),
            [pltpu.SemaphoreType.DMA for _ in range(NBUF)],
            [pltpu.SemaphoreType.DMA for _ in range(NBUF)],
        ],
    )
    def sc_segment_sum(src_hbm, dst_hbm, x_hbm, out_hbm,
                       src_v, rows, dbufs, zbuf, acc, gsems, dsems):
        cid = lax.axis_index("c")
        sid = lax.axis_index("s")
        wid = cid * NUM_SUBCORES + sid
        ebase = wid * EDGES_PER_TILE
        row0 = sid * ROWS_PER_TILE

        pltpu.sync_copy(src_hbm.at[pl.ds(ebase, EDGES_PER_TILE)], src_v)

        def start_block(b, k):
            off = pl.multiple_of(b * BLOCK_E, BLOCK_E)
            pltpu.async_copy(dst_hbm.at[pl.ds(ebase + off, BLOCK_E)],
                             dbufs[k], dsems[k])
            pltpu.async_copy(x_hbm.at[src_v.at[pl.ds(off, BLOCK_E)]],
                             rows[k], gsems[k])

        def wait_block(k):
            pltpu.make_async_copy(dst_hbm.at[pl.ds(0, BLOCK_E)],
                                  dbufs[k], dsems[k]).wait()
            pltpu.make_async_copy(x_hbm.at[src_v.at[pl.ds(0, BLOCK_E)]],
                                  rows[k], gsems[k]).wait()

        for k in range(NBUF):
            start_block(k, k)

        zvec = jnp.zeros((16,), jnp.float32)

        def zfill(i, carry):
            r = i // (D_FEAT // 16)
            col = (i % (D_FEAT // 16)) * 16
            zbuf[r, pl.ds(col, 16)] = zvec
            return carry

        lax.fori_loop(0, ZROWS * (D_FEAT // 16), zfill, 0)

        def zcopy(j, carry):
            pltpu.sync_copy(zbuf, acc.at[pl.ds(row0 + j * ZROWS, ZROWS)])
            return carry

        lax.fori_loop(0, ROWS_PER_TILE // ZROWS, zcopy, 0)

        plsc.subcore_barrier()

        main_rounds = (NUM_BLOCKS - NBUF) // NBUF

        def outer(j, carry):
            for k in range(NBUF):
                b = j * NBUF + k
                wait_block(k)
                pltpu.sync_copy(rows[k], acc.at[dbufs[k]], add=True)
                start_block(b + NBUF, k)
            return carry

        lax.fori_loop(0, main_rounds, outer, 0)

        for b in range(main_rounds * NBUF, NUM_BLOCKS):
            k = b % NBUF
            wait_block(k)
            pltpu.sync_copy(rows[k], acc.at[dbufs[k]], add=True)
            if b + NBUF < NUM_BLOCKS:
                start_block(b + NBUF, k)

        plsc.subcore_barrier()

        pltpu.sync_copy(
            acc.at[pl.ds(row0, ROWS_PER_TILE)],
            out_hbm.at[pl.ds(cid * N_PAD + row0, ROWS_PER_TILE)],
        )

    return sc_segment_sum


_SC_KERNEL = _make_sc_kernel()


def _add_body(a_ref, b_ref, o_ref):
    o_ref[...] = a_ref[...] + b_ref[...]


_ADD_BLOCK = 640
_NB_ADD = N_PAD // _ADD_BLOCK


def _combine(partials):
    summed = pl.pallas_call(
        _add_body,
        out_shape=jax.ShapeDtypeStruct((N_PAD, D_FEAT), jnp.float32),
        grid=(_NB_ADD,),
        in_specs=[
            pl.BlockSpec((_ADD_BLOCK, D_FEAT), lambda i: (i, 0)),
            pl.BlockSpec((_ADD_BLOCK, D_FEAT), lambda i: (i + _NB_ADD, 0)),
        ],
        out_specs=pl.BlockSpec((_ADD_BLOCK, D_FEAT), lambda i: (i, 0)),
    )(partials, partials)
    return summed[:N_NODES]


@jax.jit
def kernel(x, edge_index, att_w, att_b):
    del att_w, att_b
    src = edge_index[0].astype(jnp.int32)
    dst = edge_index[1].astype(jnp.int32)
    partials = _SC_KERNEL(src, dst, x.astype(jnp.float32))
    return _combine(partials)


# speedup vs baseline: 24.7207x; 1.1420x over previous
import functools

import jax
import jax.numpy as jnp
from jax import lax
from jax.experimental import pallas as pl
from jax.experimental.pallas import tpu as pltpu
from jax.experimental.pallas import tpu_sc as plsc

N_NODES = 10000
D_FEAT = 128
N_EDGES = 320000

NUM_CORES = 2
NUM_SUBCORES = 16
NUM_TILES = NUM_CORES * NUM_SUBCORES
EDGES_PER_TILE = N_EDGES // NUM_TILES
BLOCK_E = 80
NUM_BLOCKS = EDGES_PER_TILE // BLOCK_E
NBUF = 3
N_PAD = 10240
ROWS_PER_TILE = N_PAD // NUM_SUBCORES
ZROWS = 16


def _make_sc_kernel():
    mesh = plsc.VectorSubcoreMesh(core_axis_name="c", subcore_axis_name="s")

    @functools.partial(
        pl.kernel,
        mesh=mesh,
        out_type=[jax.ShapeDtypeStruct((N_PAD, D_FEAT), jnp.float32),
                  jax.ShapeDtypeStruct((N_PAD, D_FEAT), jnp.float32)],
        scratch_types=[
            pltpu.VMEM((EDGES_PER_TILE,), jnp.int32),
            [pltpu.VMEM((BLOCK_E, D_FEAT), jnp.float32) for _ in range(NBUF)],
            [pltpu.VMEM((BLOCK_E,), jnp.int32) for _ in range(NBUF)],
            pltpu.VMEM((ZROWS, D_FEAT), jnp.float32),
            pltpu.VMEM_SHARED((N_PAD, D_FEAT), jnp.float32),
            [pltpu.SemaphoreType.DMA for _ in range(NBUF)],
            [pltpu.SemaphoreType.DMA for _ in range(NBUF)],
        ],
    )
    def sc_segment_sum(edge_hbm, x_hbm, out0_hbm, out1_hbm,
                       src_v, rows, dbufs, zbuf, acc, gsems, dsems):
        cid = lax.axis_index("c")
        sid = lax.axis_index("s")
        wid = cid * NUM_SUBCORES + sid
        ebase = wid * EDGES_PER_TILE
        row0 = sid * ROWS_PER_TILE

        pltpu.sync_copy(edge_hbm.at[pl.ds(ebase, EDGES_PER_TILE)], src_v)

        def start_block(b, k):
            off = pl.multiple_of(b * BLOCK_E, BLOCK_E)
            pltpu.async_copy(edge_hbm.at[pl.ds(N_EDGES + ebase + off, BLOCK_E)],
                             dbufs[k], dsems[k])
            pltpu.async_copy(x_hbm.at[src_v.at[pl.ds(off, BLOCK_E)]],
                             rows[k], gsems[k])

        def wait_block(k):
            pltpu.make_async_copy(edge_hbm.at[pl.ds(0, BLOCK_E)],
                                  dbufs[k], dsems[k]).wait()
            pltpu.make_async_copy(x_hbm.at[src_v.at[pl.ds(0, BLOCK_E)]],
                                  rows[k], gsems[k]).wait()

        for k in range(NBUF):
            start_block(k, k)

        zvec = jnp.zeros((16,), jnp.float32)

        def zfill(i, carry):
            r = i // (D_FEAT // 16)
            col = (i % (D_FEAT // 16)) * 16
            zbuf[r, pl.ds(col, 16)] = zvec
            return carry

        lax.fori_loop(0, ZROWS * (D_FEAT // 16), zfill, 0)

        def zcopy(j, carry):
            pltpu.sync_copy(zbuf, acc.at[pl.ds(row0 + j * ZROWS, ZROWS)])
            return carry

        lax.fori_loop(0, ROWS_PER_TILE // ZROWS, zcopy, 0)

        plsc.subcore_barrier()

        main_rounds = (NUM_BLOCKS - NBUF) // NBUF

        def outer(j, carry):
            for k in range(NBUF):
                b = j * NBUF + k
                wait_block(k)
                pltpu.sync_copy(rows[k], acc.at[dbufs[k]], add=True)
                start_block(b + NBUF, k)
            return carry

        lax.fori_loop(0, main_rounds, outer, 0)

        for b in range(main_rounds * NBUF, NUM_BLOCKS):
            k = b % NBUF
            wait_block(k)
            pltpu.sync_copy(rows[k], acc.at[dbufs[k]], add=True)
            if b + NBUF < NUM_BLOCKS:
                start_block(b + NBUF, k)

        plsc.subcore_barrier()

        @pl.when(cid == 0)
        def _():
            pltpu.sync_copy(acc.at[pl.ds(row0, ROWS_PER_TILE)],
                            out0_hbm.at[pl.ds(row0, ROWS_PER_TILE)])

        @pl.when(cid == 1)
        def _():
            pltpu.sync_copy(acc.at[pl.ds(row0, ROWS_PER_TILE)],
                            out1_hbm.at[pl.ds(row0, ROWS_PER_TILE)])

    return sc_segment_sum


_SC_KERNEL = _make_sc_kernel()


def _add_body(a_ref, b_ref, o_ref):
    o_ref[...] = a_ref[...] + b_ref[...]


_ADD_BLOCK = 1000


def _combine(p0, p1):
    return pl.pallas_call(
        _add_body,
        out_shape=jax.ShapeDtypeStruct((N_NODES, D_FEAT), jnp.float32),
        grid=(N_NODES // _ADD_BLOCK,),
        in_specs=[
            pl.BlockSpec((_ADD_BLOCK, D_FEAT), lambda i: (i, 0)),
            pl.BlockSpec((_ADD_BLOCK, D_FEAT), lambda i: (i, 0)),
        ],
        out_specs=pl.BlockSpec((_ADD_BLOCK, D_FEAT), lambda i: (i, 0)),
    )(p0, p1)


@jax.jit
def kernel(x, edge_index, att_w, att_b):
    del att_w, att_b
    edge_flat = edge_index.astype(jnp.int32).reshape(-1)
    p0, p1 = _SC_KERNEL(edge_flat, x.astype(jnp.float32))
    return _combine(p0, p1)
